# baseline (speedup 1.0000x reference)
import functools

import jax
import jax.numpy as jnp
from jax import lax
from jax.experimental import pallas as pl
from jax.experimental.pallas import tpu as pltpu
from jax.experimental.pallas import tpu_sc as plsc

NC, NS, LANES = 2, 16, 16
NW = NC * NS
BATCH = 16384
EMB = 32
OUTW = 2 * EMB
BPW = BATCH // NW
NCAT = 461
NBRAND = 373

_mesh = plsc.VectorSubcoreMesh(core_axis_name="c", subcore_axis_name="s")


@functools.partial(
    pl.kernel,
    out_type=jax.ShapeDtypeStruct((BATCH * OUTW,), jnp.float32),
    mesh=_mesh,
    scratch_types=[
        pltpu.VMEM((BPW * 5,), jnp.int32),
        pltpu.VMEM((NCAT * EMB + OUTW,), jnp.float32),
        pltpu.VMEM((NBRAND * EMB + OUTW,), jnp.float32),
        pltpu.VMEM((BPW * OUTW + OUTW,), jnp.float32),
        pltpu.SemaphoreType.DMA,
    ],
    compiler_params=pltpu.CompilerParams(
        needs_layout_passes=False, use_tc_tiling_on_sc=False),
)
def _emb_kernel(fea_hbm, wcat_hbm, wbrand_hbm, out_hbm,
                fea_v, wcat_v, wbrand_v, out_v, sem):
    wid = lax.axis_index("s") * NC + lax.axis_index("c")
    base = wid * BPW

    with jax.named_scope("stage"):
        cp_cat = pltpu.make_async_copy(
            wcat_hbm, wcat_v.at[pl.ds(0, NCAT * EMB)], sem)
        cp_brand = pltpu.make_async_copy(
            wbrand_hbm, wbrand_v.at[pl.ds(0, NBRAND * EMB)], sem)
        cp_cat.start()
        cp_brand.start()
        pltpu.sync_copy(fea_hbm.at[pl.ds(base * 5, BPW * 5)], fea_v)
        cp_cat.wait()
        cp_brand.wait()

    lanes = lax.iota(jnp.int32, LANES)

    def body(t, carry):
        rows = t * LANES + lanes
        flat5 = rows * 5
        gcat = plsc.load_gather(fea_v, [flat5 + 2]) * EMB
        gbrand = plsc.load_gather(fea_v, [flat5 + 3]) * EMB
        dst = rows * OUTW
        gc = [gcat + b for b in range(8)]
        gb = [gbrand + b for b in range(8)]
        db = [dst + b for b in range(8)]
        for blk in range(4):
            vals = []
            for j in range(blk * 8, blk * 8 + 8):
                a, b = divmod(j, 8)
                vals.append((a, b, plsc.load_gather(
                    wcat_v.at[pl.ds(8 * a, NCAT * EMB)], [gc[b]])))
            for j in range(blk * 8, blk * 8 + 8):
                a, b = divmod(EMB + j, 8)
                vals.append((a, b, plsc.load_gather(
                    wbrand_v.at[pl.ds(8 * (a - 4), NBRAND * EMB)], [gb[b]])))
            for a, b, v in vals:
                plsc.store_scatter(
                    out_v.at[pl.ds(8 * a, BPW * OUTW)], [db[b]], v)
        return carry

    with jax.named_scope("assemble"):
        lax.fori_loop(0, BPW // LANES, body, 0)

    with jax.named_scope("writeout"):
        pltpu.sync_copy(out_v.at[pl.ds(0, BPW * OUTW)],
                        out_hbm.at[pl.ds(base * OUTW, BPW * OUTW)])


def kernel(item_fea, W_iid, W_title, W_cat, W_brand, W_type):
    out = _emb_kernel(item_fea.reshape(BATCH * 5),
                      W_cat.reshape(NCAT * EMB),
                      W_brand.reshape(NBRAND * EMB))
    return out.reshape(BATCH, OUTW)


# speedup vs baseline: 1.4126x; 1.4126x over previous
import functools

import jax
import jax.numpy as jnp
from jax import lax
from jax.experimental import pallas as pl
from jax.experimental.pallas import tpu as pltpu
from jax.experimental.pallas import tpu_sc as plsc

NC, NS, LANES = 2, 16, 16
NW = NC * NS
BATCH = 16384
EMB = 32
OUTW = 2 * EMB
BPW = BATCH // NW
NCAT = 461
NBRAND = 373

_mesh = plsc.VectorSubcoreMesh(core_axis_name="c", subcore_axis_name="s")


@functools.partial(
    pl.kernel,
    out_type=jax.ShapeDtypeStruct((BATCH * OUTW,), jnp.float32),
    mesh=_mesh,
    scratch_types=[
        pltpu.VMEM((BPW * 5,), jnp.int32),
        pltpu.VMEM((NCAT * EMB + OUTW,), jnp.float32),
        pltpu.VMEM((NBRAND * EMB + OUTW,), jnp.float32),
        pltpu.VMEM((BPW * OUTW + OUTW,), jnp.float32),
        pltpu.SemaphoreType.DMA,
    ],
    compiler_params=pltpu.CompilerParams(
        needs_layout_passes=False, use_tc_tiling_on_sc=False),
)
def _emb_kernel(fea_hbm, wcat_hbm, wbrand_hbm, out_hbm,
                fea_v, wcat_v, wbrand_v, out_v, sem):
    wid = lax.axis_index("s") * NC + lax.axis_index("c")
    base = wid * BPW

    with jax.named_scope("stage"):
        cp_cat = pltpu.make_async_copy(
            wcat_hbm, wcat_v.at[pl.ds(0, NCAT * EMB)], sem)
        cp_brand = pltpu.make_async_copy(
            wbrand_hbm, wbrand_v.at[pl.ds(0, NBRAND * EMB)], sem)
        cp_cat.start()
        cp_brand.start()
        pltpu.sync_copy(fea_hbm.at[pl.ds(base * 5, BPW * 5)], fea_v)
        cp_cat.wait()
        cp_brand.wait()

    lanes = lax.iota(jnp.int32, LANES)
    rot = [(lanes + i) & 15 for i in range(LANES)]

    def body(t, carry):
        rows = t * LANES + lanes
        flat5 = rows * 5
        gcat = plsc.load_gather(fea_v, [flat5 + 2]) * EMB
        gbrand = plsc.load_gather(fea_v, [flat5 + 3]) * EMB
        dst = rows * OUTW
        for blk in range(2):
            coff = 16 * blk
            for half in range(2):
                vals = []
                for i in range(half * 8, half * 8 + 8):
                    vals.append((i, plsc.load_gather(
                        wcat_v.at[pl.ds(coff, NCAT * EMB)], [gcat + rot[i]])))
                for i in range(half * 8, half * 8 + 8):
                    vals.append((LANES + i, plsc.load_gather(
                        wbrand_v.at[pl.ds(coff, NBRAND * EMB)],
                        [gbrand + rot[i]])))
                for k, v in vals:
                    if k < LANES:
                        plsc.store_scatter(
                            out_v.at[pl.ds(coff, BPW * OUTW)],
                            [dst + rot[k]], v)
                    else:
                        plsc.store_scatter(
                            out_v.at[pl.ds(EMB + coff, BPW * OUTW)],
                            [dst + rot[k - LANES]], v)
        return carry

    with jax.named_scope("assemble"):
        lax.fori_loop(0, BPW // LANES, body, 0)

    with jax.named_scope("writeout"):
        pltpu.sync_copy(out_v.at[pl.ds(0, BPW * OUTW)],
                        out_hbm.at[pl.ds(base * OUTW, BPW * OUTW)])


def kernel(item_fea, W_iid, W_title, W_cat, W_brand, W_type):
    out = _emb_kernel(item_fea.reshape(BATCH * 5),
                      W_cat.reshape(NCAT * EMB),
                      W_brand.reshape(NBRAND * EMB))
    return out.reshape(BATCH, OUTW)
